# baseline (speedup 1.0000x reference)
import functools

import jax
import jax.numpy as jnp
from jax import lax
from jax.experimental import pallas as pl
from jax.experimental.pallas import tpu as pltpu
from jax.experimental.pallas import tpu_sc as plsc

NUM_EMB = 1_000_000
DIM = 64

NC = 2
NS = 16
NW = NC * NS

COLS_HALF = 13
ROWS = 16384 * COLS_HALF
PER_W = ROWS // NW
SUB = 128
SUBS_PER_CHUNK = 4
CHUNK = SUB * SUBS_PER_CHUNK
N_CHUNKS = PER_W // CHUNK
NBUF = 2


def _make_kernel():
    mesh = plsc.VectorSubcoreMesh(
        core_axis_name="c", subcore_axis_name="s",
        num_cores=NC, num_subcores=NS)

    @functools.partial(
        pl.kernel,
        out_type=jax.ShapeDtypeStruct((ROWS, DIM), jnp.float32),
        mesh=mesh,
        scratch_types=[
            pltpu.VMEM((PER_W,), jnp.int32),
            pltpu.VMEM((NBUF, CHUNK, DIM), jnp.float32),
            [pltpu.SemaphoreType.DMA] * NBUF,
            [pltpu.SemaphoreType.DMA] * NBUF,
        ],
        compiler_params=pltpu.CompilerParams(
            use_tc_tiling_on_sc=False,
        ),
    )
    def gather_kernel(idx_hbm, table_hbm, out_hbm, idx_v, rows_v, gsem, ssem):
        wid = lax.axis_index("s") * NC + lax.axis_index("c")
        base = wid * PER_W
        pltpu.sync_copy(idx_hbm.at[pl.ds(base, PER_W)], idx_v)

        def fire_gathers(b, c):
            return [
                pltpu.async_copy(
                    table_hbm.at[idx_v.at[pl.ds(c * CHUNK + j * SUB, SUB)]],
                    rows_v.at[b, pl.ds(j * SUB, SUB)],
                    gsem[b])
                for j in range(SUBS_PER_CHUNK)
            ]

        def fire_store(b, c):
            return pltpu.async_copy(
                rows_v.at[b], out_hbm.at[pl.ds(base + c * CHUNK, CHUNK)],
                ssem[b])

        gather_h = [None] * NBUF
        store_h = [None] * NBUF
        for c in range(N_CHUNKS):
            b = c % NBUF
            if store_h[b] is not None:
                store_h[b].wait()
            gather_h[b] = fire_gathers(b, c)
            if c >= 1:
                pb = (c - 1) % NBUF
                for h in gather_h[pb]:
                    h.wait()
                store_h[pb] = fire_store(pb, c - 1)
        lb = (N_CHUNKS - 1) % NBUF
        for h in gather_h[lb]:
            h.wait()
        store_h[lb] = fire_store(lb, N_CHUNKS - 1)
        for b in range(NBUF):
            store_h[b].wait()

    return gather_kernel


_gather = _make_kernel()


def kernel(x, table):
    xi = x.astype(jnp.int32)
    o1 = _gather(xi[:, :COLS_HALF].reshape(-1), table)
    o2 = _gather(xi[:, COLS_HALF:].reshape(-1), table)
    b = x.shape[0]
    return jnp.concatenate(
        [o1.reshape(b, COLS_HALF, DIM), o2.reshape(b, COLS_HALF, DIM)],
        axis=1)


# speedup vs baseline: 1.0454x; 1.0454x over previous
import functools

import jax
import jax.numpy as jnp
from jax import lax
from jax.experimental import pallas as pl
from jax.experimental.pallas import tpu as pltpu
from jax.experimental.pallas import tpu_sc as plsc

NUM_EMB = 1_000_000
DIM = 64

NC = 2
NS = 16
NW = NC * NS

ROWS = 16384 * 26
PER_W = ROWS // NW
SUB = 128
SUBS_PER_CHUNK = 4
CHUNK = SUB * SUBS_PER_CHUNK
N_CHUNKS = PER_W // CHUNK
NBUF = 2


def _make_kernel():
    mesh = plsc.VectorSubcoreMesh(
        core_axis_name="c", subcore_axis_name="s",
        num_cores=NC, num_subcores=NS)

    @functools.partial(
        pl.kernel,
        out_type=jax.ShapeDtypeStruct((ROWS, DIM), jnp.float32),
        mesh=mesh,
        scratch_types=[
            pltpu.VMEM((PER_W,), jnp.int32),
            pltpu.VMEM((NBUF, CHUNK, DIM), jnp.float32),
            [pltpu.SemaphoreType.DMA] * NBUF,
            [pltpu.SemaphoreType.DMA] * NBUF,
        ],
        compiler_params=pltpu.CompilerParams(
            use_tc_tiling_on_sc=False,
        ),
    )
    def gather_kernel(idx_hbm, table_hbm, out_hbm, idx_v, rows_v, gsem, ssem):
        wid = lax.axis_index("s") * NC + lax.axis_index("c")
        base = wid * PER_W
        pltpu.sync_copy(idx_hbm.at[pl.ds(base, PER_W)], idx_v)

        def fire_gathers(b, c):
            return [
                pltpu.async_copy(
                    table_hbm.at[idx_v.at[pl.ds(c * CHUNK + j * SUB, SUB)]],
                    rows_v.at[b, pl.ds(j * SUB, SUB)],
                    gsem[b])
                for j in range(SUBS_PER_CHUNK)
            ]

        def fire_store(b, c):
            return pltpu.async_copy(
                rows_v.at[b], out_hbm.at[pl.ds(base + c * CHUNK, CHUNK)],
                ssem[b])

        gather_h = [None] * NBUF
        store_h = [None] * NBUF
        for c in range(N_CHUNKS):
            b = c % NBUF
            if store_h[b] is not None:
                store_h[b].wait()
            gather_h[b] = fire_gathers(b, c)
            if c >= 1:
                pb = (c - 1) % NBUF
                for h in gather_h[pb]:
                    h.wait()
                store_h[pb] = fire_store(pb, c - 1)
        lb = (N_CHUNKS - 1) % NBUF
        for h in gather_h[lb]:
            h.wait()
        store_h[lb] = fire_store(lb, N_CHUNKS - 1)
        for b in range(NBUF):
            store_h[b].wait()

    return gather_kernel


_gather = _make_kernel()


def kernel(x, table):
    flat = x.reshape(-1).astype(jnp.int32)
    out = _gather(flat, table)
    return out.reshape(x.shape + (DIM,))


# speedup vs baseline: 1.0906x; 1.0433x over previous
import functools

import jax
import jax.numpy as jnp
from jax import lax
from jax.experimental import pallas as pl
from jax.experimental.pallas import tpu as pltpu
from jax.experimental.pallas import tpu_sc as plsc

NUM_EMB = 1_000_000
DIM = 64

NC = 2
NS = 16
NW = NC * NS

ROWS = 16384 * 26
PER_W = ROWS // NW
SUB = 128
SUBS_PER_CHUNK = 4
CHUNK = SUB * SUBS_PER_CHUNK
N_CHUNKS = PER_W // CHUNK
NBUF = 2


def _make_kernel():
    mesh = plsc.VectorSubcoreMesh(
        core_axis_name="c", subcore_axis_name="s",
        num_cores=NC, num_subcores=NS)

    @functools.partial(
        pl.kernel,
        out_type=jax.ShapeDtypeStruct((ROWS, DIM), jnp.float32),
        mesh=mesh,
        scratch_types=[
            pltpu.VMEM((PER_W,), jnp.int32),
            pltpu.VMEM((NBUF, CHUNK, DIM), jnp.float32),
            [pltpu.SemaphoreType.DMA] * NBUF,
            [pltpu.SemaphoreType.DMA] * NBUF,
        ],
        compiler_params=pltpu.CompilerParams(
            use_tc_tiling_on_sc=False,
        ),
    )
    def gather_kernel(idx_hbm, table_hbm, out_hbm, idx_v, rows_v, gsem, ssem):
        wid = lax.axis_index("s") * NC + lax.axis_index("c")
        base = wid * PER_W
        pltpu.sync_copy(idx_hbm.at[pl.ds(base, PER_W)], idx_v)

        def fire_gathers(b, c):
            return [
                pltpu.async_copy(
                    table_hbm.at[idx_v.at[pl.ds(c * CHUNK + j * SUB, SUB)]],
                    rows_v.at[b, pl.ds(j * SUB, SUB)],
                    gsem[b])
                for j in range(SUBS_PER_CHUNK)
            ]

        def fire_store(b, c):
            return pltpu.async_copy(
                rows_v.at[b], out_hbm.at[pl.ds(base + c * CHUNK, CHUNK)],
                ssem[b])

        gather_h = [None] * NBUF
        store_h = [None] * NBUF
        for c in range(N_CHUNKS):
            b = c % NBUF
            if store_h[b] is not None:
                store_h[b].wait()
            gather_h[b] = fire_gathers(b, c)
            if c >= 1:
                pb = (c - 1) % NBUF
                for h in gather_h[pb]:
                    h.wait()
                store_h[pb] = fire_store(pb, c - 1)
        lb = (N_CHUNKS - 1) % NBUF
        for h in gather_h[lb]:
            h.wait()
        store_h[lb] = fire_store(lb, N_CHUNKS - 1)
        for b in range(NBUF):
            store_h[b].wait()

    return gather_kernel


_gather = _make_kernel()


def kernel(x, table):
    flat = x.T.reshape(-1).astype(jnp.int32)
    out = _gather(flat, table)
    return out.reshape(x.shape[1], x.shape[0], DIM).transpose(1, 0, 2)
